# baseline (speedup 1.0000x reference)
import jax
import jax.numpy as jnp
from jax.experimental import pallas as pl
from jax.experimental.pallas import tpu as pltpu
from functools import partial

_P = 0.1
_VOCAB = 1000
_SEED = 42

_BR = 256


def _sym_channel_kernel(msg_ref, tgt_ref, rep_ref, out_ref):
    m = msg_ref[...]
    mx = jnp.max(m, axis=1, keepdims=True)
    lane = jax.lax.broadcasted_iota(jnp.int32, m.shape, 1)
    idx = jnp.min(jnp.where(m == mx, lane, jnp.int32(2**30)),
                  axis=1, keepdims=True)
    msg_exp = jnp.maximum(idx, 1)
    rep = rep_ref[...]
    repl_sym = jnp.where(rep + 1 < msg_exp, rep + 1, rep + 2)
    combined = (tgt_ref[...] != 0) & (idx != 0)
    onehot = (lane == repl_sym).astype(m.dtype)
    out_ref[...] = jnp.where(combined, onehot, m)


@partial(jax.jit, static_argnames=())
def kernel(message, apply_noise):
    B, L, V = message.shape
    n = B * L
    flat = message.reshape(n, V)

    key = jax.random.key(_SEED)
    k1, k2 = jax.random.split(key)
    tgt = jax.random.uniform(k1, (B, L)) < _P
    rep = jax.random.randint(k2, (B, L), 0, _VOCAB - 2).astype(jnp.int32)
    tgt_eff = jnp.logical_and(tgt, apply_noise != 0)

    tgt_col = tgt_eff.reshape(n, 1).astype(jnp.int32)
    rep_col = rep.reshape(n, 1)

    grid = (n // _BR,)
    out = pl.pallas_call(
        _sym_channel_kernel,
        grid=grid,
        in_specs=[
            pl.BlockSpec((_BR, V), lambda i: (i, 0)),
            pl.BlockSpec((_BR, 1), lambda i: (i, 0)),
            pl.BlockSpec((_BR, 1), lambda i: (i, 0)),
        ],
        out_specs=pl.BlockSpec((_BR, V), lambda i: (i, 0)),
        out_shape=jax.ShapeDtypeStruct((n, V), message.dtype),
    )(flat, tgt_col, rep_col)
    return out.reshape(B, L, V)
